# baseline (speedup 1.0000x reference)
import functools

import jax
import jax.numpy as jnp
from jax import lax
from jax.experimental import pallas as pl
from jax.experimental.pallas import tpu as pltpu
from jax.experimental.pallas import tpu_sc as plsc

N_U = 25000
N_I = 25000
N = N_U + N_I
EMB = 64
N_LAYERS = 3
E = 800000

NP = 50048
N_WORKERS = 32
RPT = NP // N_WORKERS
ACCW = RPT * EMB

SCAN = 1024
NSCAN = 784
EP = NSCAN * SCAN
B = 192
BG = B // 16
PEND = 2048
PENDA = 2176
SUB = 512
SUBCAP = 448
NBX = (EP + N_WORKERS * B) // B + 2
EPX = NBX * B

_mesh = plsc.VectorSubcoreMesh(core_axis_name="c", subcore_axis_name="s")
_params = pltpu.CompilerParams(use_tc_tiling_on_sc=False,
                               needs_layout_passes=False)


def _region(cstage, c, s):
    iota = lax.iota(jnp.int32, 16)
    c0 = plsc.load_gather(cstage, [iota * 8])
    c1 = plsc.load_gather(cstage, [iota * 8 + 128])
    p0 = ((c0 + (B - 1)) // B) * B
    p1 = ((c1 + (B - 1)) // B) * B
    i0 = jnp.cumsum(p0)
    i1 = jnp.cumsum(p1) + jnp.full((16,), i0[15], jnp.int32)
    e0 = i0 - p0
    e1 = i1 - p1
    is0 = jnp.full((16,), c, jnp.int32) == 0
    esel = jnp.where(is0, e0, e1)
    psel = jnp.where(is0, p0, p1)
    sv = jnp.full((16,), s, jnp.int32)
    return jnp.take(esel, sv)[0], jnp.take(psel, sv)[0]


@functools.partial(
    pl.kernel,
    mesh=_mesh,
    out_type=jax.ShapeDtypeStruct((N_WORKERS * 8,), jnp.int32),
    scratch_types=[
        pltpu.VMEM((SCAN,), jnp.int32),
        pltpu.VMEM((SCAN,), jnp.int32),
        pltpu.VMEM((SCAN,), jnp.int32),
        pltpu.VMEM((SCAN,), jnp.int32),
        pltpu.VMEM((16,), jnp.int32),
        pltpu.SemaphoreType.DMA,
        pltpu.SemaphoreType.DMA,
        pltpu.SemaphoreType.DMA,
        pltpu.SemaphoreType.DMA,
    ],
    compiler_params=_params,
)
def _count(dstE, out, d0, d1, d2, d3, cbuf, s0, s1, s2, s3):
    c = lax.axis_index("c")
    s = lax.axis_index("s")
    g = c * 16 + s
    lo = g * RPT
    hi = lo + RPT
    bufs = [(d0, s0), (d1, s1), (d2, s2), (d3, s3)]

    def stage_start(m, dbuf, sem):
        off = pl.multiple_of(jnp.minimum(m, NSCAN - 1) * SCAN, SCAN)
        pltpu.make_async_copy(dstE.at[pl.ds(off, SCAN)], dbuf, sem).start()

    def stage_wait(dbuf, sem):
        pltpu.make_async_copy(dstE.at[pl.ds(0, SCAN)], dbuf, sem).wait()

    for j in range(4):
        stage_start(j, *bufs[j])

    def scan_chunk(dbuf, cntv):
        def scan_step(i, cntv):
            d = dbuf[pl.ds(i * 16, 16)]
            msk = (d >= lo) & (d < hi)
            return cntv + jnp.where(msk, 1, 0)

        return lax.fori_loop(0, SCAN // 16, scan_step, cntv, unroll=8)

    def quad_body(q, cntv):
        m = q * 4
        for j in range(4):
            dbuf, sem = bufs[j]
            stage_wait(dbuf, sem)
            cntv = scan_chunk(dbuf, cntv)
            stage_start(m + 4 + j, dbuf, sem)
        return cntv

    cntv = lax.fori_loop(0, NSCAN // 4, quad_body, jnp.zeros((16,), jnp.int32))
    for j in range(4):
        stage_wait(*bufs[j])
    total = jnp.sum(cntv)
    cbuf[...] = jnp.full((16,), total, jnp.int32)
    pltpu.sync_copy(cbuf.at[pl.ds(0, 8)],
                    out.at[pl.ds(pl.multiple_of(g * 8, 8), 8)])


@functools.partial(
    pl.kernel,
    mesh=_mesh,
    out_type=jax.ShapeDtypeStruct((3 * EPX,), jnp.int32),
    scratch_types=[
        pltpu.VMEM((SCAN,), jnp.int32),
        pltpu.VMEM((SCAN,), jnp.int32),
        pltpu.VMEM((SCAN,), jnp.int32),
        pltpu.VMEM((SCAN,), jnp.int32),
        pltpu.VMEM((PENDA,), jnp.int32),
        pltpu.VMEM((PENDA,), jnp.int32),
        pltpu.VMEM((B,), jnp.int32),
        pltpu.VMEM((B,), jnp.float32),
        pltpu.VMEM((3 * B,), jnp.int32),
        pltpu.VMEM((N_WORKERS * 8,), jnp.int32),
        pltpu.SemaphoreType.DMA,
        pltpu.SemaphoreType.DMA,
        pltpu.SemaphoreType.DMA,
        pltpu.SemaphoreType.DMA,
        pltpu.SemaphoreType.DMA,
        pltpu.SemaphoreType.DMA,
    ],
    compiler_params=_params,
)
def _extract(dstE, srcE, wgtE, counts, recsP,
             d0, d1, d2, d3, idbuf, basebuf, sstage, wstage, rec, cstage,
             s0, s1, s2, s3, sem, sem2):
    c = lax.axis_index("c")
    s = lax.axis_index("s")
    g = c * 16 + s
    lo = g * RPT
    hi = lo + RPT

    pltpu.sync_copy(counts, cstage)
    myoff, _ = _region(cstage, c, s)

    iota = lax.iota(jnp.int32, 16)
    bufs = [(d0, s0), (d1, s1), (d2, s2), (d3, s3)]

    def stage_start(m, dbuf, dsem):
        off = pl.multiple_of(jnp.minimum(m, NSCAN - 1) * SCAN, SCAN)
        pltpu.make_async_copy(dstE.at[pl.ds(off, SCAN)], dbuf, dsem).start()

    def stage_wait(dbuf, dsem):
        pltpu.make_async_copy(dstE.at[pl.ds(0, SCAN)], dbuf, dsem).wait()

    for j in range(4):
        stage_start(j, *bufs[j])

    def flush(src_off, dst_batch_off):
        for j in range(BG):
            rec[pl.ds(2 * B + j * 16, 16)] = basebuf[pl.ds(src_off + j * 16, 16)]
            sstage[pl.ds(j * 16, 16)] = idbuf[pl.ds(src_off + j * 16, 16)]
        h1 = pltpu.async_copy(srcE.at[sstage], rec.at[pl.ds(0, B)], sem)
        h2 = pltpu.async_copy(wgtE.at[sstage], wstage, sem2)
        h1.wait()
        h2.wait()
        for j in range(BG):
            rec[pl.ds(B + j * 16, 16)] = plsc.bitcast(
                wstage[pl.ds(j * 16, 16)], jnp.int32)
        pltpu.sync_copy(rec, recsP.at[pl.ds(dst_batch_off, 3 * B)])

    def do_chunk(m, dscan, carry):
        c0, c1, c2, c3, nbw = carry
        cbase = m * SCAN

        def scan_step(i, cnts):
            new = []
            for r in range(4):
                cnt = cnts[r]
                st = i * 4 + r
                d = dscan[pl.ds(st * 16, 16)]
                msk = (d >= lo) & (d < hi)
                ids = jnp.full((16,), cbase + st * 16, dtype=jnp.int32) + iota
                base = (d - lo) * EMB
                incl = jnp.cumsum(jnp.where(msk, 1, 0))
                pos = jnp.where(msk, r * SUB + cnt + incl - 1,
                                r * SUB + SUBCAP + iota)
                plsc.store_scatter(idbuf, [pos], ids)
                plsc.store_scatter(basebuf, [pos], base)
                new.append(cnt + incl[15])
            return tuple(new)

        c0, c1, c2, c3 = lax.fori_loop(0, SCAN // 64, scan_step,
                                       (c0, c1, c2, c3))

        cnts = [c0, c1, c2, c3]
        outc = []
        for r in range(4):
            cnt = cnts[r]
            nb = cnt // B

            def flush_body(b, nbw, r=r):
                dst_off = pl.multiple_of((myoff // B + nbw) * 3 * B, 3 * B)
                flush(r * SUB + b * B, dst_off)
                return nbw + 1

            nbw = lax.fori_loop(0, nb, flush_body, nbw)

            rem_start = r * SUB + nb * B
            for j in range(BG):
                tv = idbuf[pl.ds(rem_start + j * 16, 16)]
                bv = basebuf[pl.ds(rem_start + j * 16, 16)]
                idbuf[pl.ds(r * SUB + j * 16, 16)] = tv
                basebuf[pl.ds(r * SUB + j * 16, 16)] = bv
            outc.append(cnt - nb * B)
        return (outc[0], outc[1], outc[2], outc[3], nbw)

    def quad_body(q, carry):
        m = q * 4
        for j in range(4):
            dbuf, dsem = bufs[j]
            stage_wait(dbuf, dsem)
            carry = do_chunk(m + j, dbuf, carry)
            stage_start(m + 4 + j, dbuf, dsem)
        return carry

    c0, c1, c2, c3, nbw = lax.fori_loop(0, NSCAN // 4, quad_body,
                                        (0, 0, 0, 0, 0))
    for j in range(4):
        stage_wait(*bufs[j])

    merged = c0
    for r in range(1, 4):
        for j in range(BG):
            tv = idbuf[pl.ds(r * SUB + j * 16, 16)]
            bv = basebuf[pl.ds(r * SUB + j * 16, 16)]
            idbuf[pl.ds(merged + j * 16, 16)] = tv
            basebuf[pl.ds(merged + j * 16, 16)] = bv
        merged = merged + [c1, c2, c3][r - 1]

    nbm = merged // B

    def mflush_body(b, nbw):
        dst_off = pl.multiple_of((myoff // B + nbw) * 3 * B, 3 * B)
        flush(b * B, dst_off)
        return nbw + 1

    nbw = lax.fori_loop(0, nbm, mflush_body, nbw)
    cnt = merged - nbm * B

    rem_start = nbm * B
    for j in range(BG):
        tv = idbuf[pl.ds(rem_start + j * 16, 16)]
        bv = basebuf[pl.ds(rem_start + j * 16, 16)]
        idbuf[pl.ds(j * 16, 16)] = tv
        basebuf[pl.ds(j * 16, 16)] = bv

    @pl.when(cnt > 0)
    def _():
        sent_id = jnp.full((16,), EP - 1, dtype=jnp.int32)
        sent_base = jnp.zeros((16,), jnp.int32)
        for j in range(BG):
            idbuf[pl.ds(cnt + j * 16, 16)] = sent_id
            basebuf[pl.ds(cnt + j * 16, 16)] = sent_base
        flush(0, pl.multiple_of((myoff // B + nbw) * 3 * B, 3 * B))

    @pl.when(g == N_WORKERS - 1)
    def _():
        sent_id = jnp.full((16,), EP - 1, dtype=jnp.int32)
        sent_base = jnp.zeros((16,), jnp.int32)
        for j in range(BG):
            idbuf[pl.ds(j * 16, 16)] = sent_id
            basebuf[pl.ds(j * 16, 16)] = sent_base
        mycp_end = myoff // B + nbw + jnp.where(cnt > 0, 1, 0)
        flush(0, pl.multiple_of(mycp_end * 3 * B, 3 * B))
        flush(0, pl.multiple_of((mycp_end + 1) * 3 * B, 3 * B))


@functools.partial(
    pl.kernel,
    mesh=_mesh,
    out_type=jax.ShapeDtypeStruct((NP * EMB,), jnp.float32),
    scratch_types=[
        pltpu.VMEM((B,), jnp.int32),
        pltpu.VMEM((B,), jnp.int32),
        pltpu.VMEM((B,), jnp.int32),
        pltpu.VMEM((B,), jnp.int32),
        pltpu.VMEM((2 * B,), jnp.int32),
        pltpu.VMEM((2 * B,), jnp.int32),
        pltpu.VMEM((B, EMB), jnp.float32),
        pltpu.VMEM((B, EMB), jnp.float32),
        pltpu.VMEM((N_WORKERS * 8,), jnp.int32),
        pltpu.VMEM((ACCW,), jnp.float32),
        pltpu.SemaphoreType.DMA,
        pltpu.SemaphoreType.DMA,
        pltpu.SemaphoreType.DMA,
        pltpu.SemaphoreType.DMA,
    ],
    compiler_params=_params,
)
def _layer(tbl, recsP, counts, out,
           src_a, src_b, gsrc_a, gsrc_b, wb_a, wb_b, rows_a, rows_b,
           cstage, acc, sem_ra, sem_rb, sem_ga, sem_gb):
    c = lax.axis_index("c")
    s = lax.axis_index("s")
    g = c * 16 + s

    pltpu.sync_copy(counts, cstage)
    myoff, mycp = _region(cstage, c, s)
    b0 = myoff // B
    nb = mycp // B

    iota = lax.iota(jnp.int32, 16)
    zero16 = jnp.zeros((16,), jnp.float32)

    def zero_body(i, _):
        acc[pl.ds(i * 16, 16)] = zero16
        return 0

    lax.fori_loop(0, ACCW // 16, zero_body, 0, unroll=8)

    def start_recs(bi, sbuf, wbuf, sem):
        off = pl.multiple_of((b0 + bi) * 3 * B, 3 * B)
        cp1 = pltpu.make_async_copy(recsP.at[pl.ds(off, B)], sbuf, sem)
        cp2 = pltpu.make_async_copy(
            recsP.at[pl.ds(pl.multiple_of(off + B, 8), 2 * B)], wbuf, sem)
        cp1.start()
        cp2.start()

    def wait_recs(sbuf, wbuf, sem):
        pltpu.make_async_copy(recsP.at[pl.ds(0, B)], sbuf, sem).wait()
        pltpu.make_async_copy(recsP.at[pl.ds(0, 2 * B)], wbuf, sem).wait()

    def start_rows(sbuf, rbuf, sem):
        pltpu.make_async_copy(tbl.at[sbuf], rbuf, sem).start()

    def wait_rows(sbuf, rbuf, sem):
        pltpu.make_async_copy(tbl.at[sbuf], rbuf, sem).wait()

    def compute(wbuf, rbuf):
        def acc_body(q, _):
            e0 = q * 16
            w16 = plsc.bitcast(wbuf[pl.ds(e0, 16)], jnp.float32)
            b16 = wbuf[pl.ds(B + e0, 16)]
            for l in range(16):
                e = e0 + l
                wv = jnp.full((16,), w16[l], dtype=jnp.float32)
                bv = jnp.full((16,), b16[l], dtype=jnp.int32) + iota
                xs = [rbuf[e, pl.ds(j * 16, 16)] * wv for j in range(4)]
                ix = [bv + (j * 16) for j in range(4)]
                for j in range(4):
                    plsc.addupdate_scatter(acc, [ix[j]], xs[j])
            return 0

        lax.fori_loop(0, BG, acc_body, 0, unroll=2)

    def snap(sbuf, gbuf):
        for j in range(BG):
            gbuf[pl.ds(j * 16, 16)] = sbuf[pl.ds(j * 16, 16)]

    start_recs(0, src_a, wb_a, sem_ra)
    start_recs(1, src_b, wb_b, sem_rb)
    wait_recs(src_a, wb_a, sem_ra)
    snap(src_a, gsrc_a)
    start_rows(gsrc_a, rows_a, sem_ga)

    def pair_body(k, _):
        ba = 2 * k
        wait_recs(src_b, wb_b, sem_rb)
        snap(src_b, gsrc_b)
        start_rows(gsrc_b, rows_b, sem_gb)
        wait_rows(gsrc_a, rows_a, sem_ga)
        compute(wb_a, rows_a)
        start_recs(ba + 2, src_a, wb_a, sem_ra)
        wait_recs(src_a, wb_a, sem_ra)
        snap(src_a, gsrc_a)
        start_rows(gsrc_a, rows_a, sem_ga)
        wait_rows(gsrc_b, rows_b, sem_gb)
        compute(wb_b, rows_b)
        start_recs(ba + 3, src_b, wb_b, sem_rb)
        return 0

    lax.fori_loop(0, nb // 2, pair_body, 0)

    wait_rows(gsrc_a, rows_a, sem_ga)

    @pl.when(nb % 2 == 1)
    def _():
        compute(wb_a, rows_a)

    wait_recs(src_b, wb_b, sem_rb)

    pltpu.sync_copy(acc, out.at[pl.ds(pl.multiple_of(g * ACCW, 8), ACCW)])


_MEAN_BLK = 3128


def _mean_body(t0, t1, t2, t3, o):
    o[...] = 0.25 * ((t0[...] + t1[...]) + (t2[...] + t3[...]))


_mean = pl.pallas_call(
    _mean_body,
    grid=(NP // _MEAN_BLK,),
    in_specs=[pl.BlockSpec((_MEAN_BLK, EMB), lambda i: (i, 0))] * 4,
    out_specs=pl.BlockSpec((_MEAN_BLK, EMB), lambda i: (i, 0)),
    out_shape=jax.ShapeDtypeStruct((NP, EMB), jnp.float32),
)


def kernel(edge_index, edge_weight, user_emb, item_emb):
    rpad = jnp.zeros((NP - N, EMB), jnp.float32)
    t0 = jnp.concatenate([user_emb, item_emb, rpad], axis=0)

    pad = EP - E
    pad_idx = (jnp.arange(pad, dtype=jnp.int32) * 97) % N
    srcE = jnp.concatenate([edge_index[0], pad_idx])
    dstE = jnp.concatenate([edge_index[1], pad_idx])
    wgtE = jnp.concatenate([edge_weight, jnp.zeros((pad,), jnp.float32)])

    counts = _count(dstE)
    recsP = _extract(dstE, srcE, wgtE, counts)

    t1 = _layer(t0, recsP, counts).reshape(NP, EMB)
    t2 = _layer(t1, recsP, counts).reshape(NP, EMB)
    t3 = _layer(t2, recsP, counts).reshape(NP, EMB)

    final = _mean(t0, t1, t2, t3)
    return final[:N_U], final[N_U:N]


# speedup vs baseline: 1.3679x; 1.3679x over previous
import functools

import jax
import jax.numpy as jnp
from jax import lax
from jax.experimental import pallas as pl
from jax.experimental.pallas import tpu as pltpu
from jax.experimental.pallas import tpu_sc as plsc

N_U = 25000
N_I = 25000
N = N_U + N_I
EMB = 64
N_LAYERS = 3
E = 800000

NP = 50048
N_WORKERS = 32
RPT = NP // N_WORKERS
ACCW = RPT * EMB

SCAN = 1024
NSCAN = 784
EP = NSCAN * SCAN
B = 192
BG = B // 16
PEND = 2048
PENDA = PEND + 16
NBX = (EP + N_WORKERS * B) // B + 2
EPX = NBX * B

_mesh = plsc.VectorSubcoreMesh(core_axis_name="c", subcore_axis_name="s")
_params = pltpu.CompilerParams(use_tc_tiling_on_sc=False,
                               needs_layout_passes=False)


def _region(cstage, c, s):
    iota = lax.iota(jnp.int32, 16)
    c0 = plsc.load_gather(cstage, [iota * 8])
    c1 = plsc.load_gather(cstage, [iota * 8 + 128])
    p0 = ((c0 + (B - 1)) // B) * B
    p1 = ((c1 + (B - 1)) // B) * B
    i0 = jnp.cumsum(p0)
    i1 = jnp.cumsum(p1) + jnp.full((16,), i0[15], jnp.int32)
    e0 = i0 - p0
    e1 = i1 - p1
    is0 = jnp.full((16,), c, jnp.int32) == 0
    esel = jnp.where(is0, e0, e1)
    psel = jnp.where(is0, p0, p1)
    sv = jnp.full((16,), s, jnp.int32)
    return jnp.take(esel, sv)[0], jnp.take(psel, sv)[0]


@functools.partial(
    pl.kernel,
    mesh=_mesh,
    out_type=jax.ShapeDtypeStruct((N_WORKERS * 8,), jnp.int32),
    scratch_types=[
        pltpu.VMEM((SCAN,), jnp.int32),
        pltpu.VMEM((SCAN,), jnp.int32),
        pltpu.VMEM((SCAN,), jnp.int32),
        pltpu.VMEM((SCAN,), jnp.int32),
        pltpu.VMEM((16,), jnp.int32),
        pltpu.SemaphoreType.DMA,
        pltpu.SemaphoreType.DMA,
        pltpu.SemaphoreType.DMA,
        pltpu.SemaphoreType.DMA,
    ],
    compiler_params=_params,
)
def _count(dstE, out, d0, d1, d2, d3, cbuf, s0, s1, s2, s3):
    c = lax.axis_index("c")
    s = lax.axis_index("s")
    g = c * 16 + s
    lo = g * RPT
    hi = lo + RPT
    bufs = [(d0, s0), (d1, s1), (d2, s2), (d3, s3)]

    def stage_start(m, dbuf, sem):
        off = pl.multiple_of(jnp.minimum(m, NSCAN - 1) * SCAN, SCAN)
        pltpu.make_async_copy(dstE.at[pl.ds(off, SCAN)], dbuf, sem).start()

    def stage_wait(dbuf, sem):
        pltpu.make_async_copy(dstE.at[pl.ds(0, SCAN)], dbuf, sem).wait()

    for j in range(4):
        stage_start(j, *bufs[j])

    def scan_chunk(dbuf, cntv):
        def scan_step(i, cntv):
            d = dbuf[pl.ds(i * 16, 16)]
            msk = (d >= lo) & (d < hi)
            return cntv + jnp.where(msk, 1, 0)

        return lax.fori_loop(0, SCAN // 16, scan_step, cntv, unroll=8)

    def quad_body(q, cntv):
        m = q * 4
        for j in range(4):
            dbuf, sem = bufs[j]
            stage_wait(dbuf, sem)
            cntv = scan_chunk(dbuf, cntv)
            stage_start(m + 4 + j, dbuf, sem)
        return cntv

    cntv = lax.fori_loop(0, NSCAN // 4, quad_body, jnp.zeros((16,), jnp.int32))
    for j in range(4):
        stage_wait(*bufs[j])
    total = jnp.sum(cntv)
    cbuf[...] = jnp.full((16,), total, jnp.int32)
    pltpu.sync_copy(cbuf.at[pl.ds(0, 8)],
                    out.at[pl.ds(pl.multiple_of(g * 8, 8), 8)])


@functools.partial(
    pl.kernel,
    mesh=_mesh,
    out_type=jax.ShapeDtypeStruct((3 * EPX,), jnp.int32),
    scratch_types=[
        pltpu.VMEM((SCAN,), jnp.int32),
        pltpu.VMEM((SCAN,), jnp.int32),
        pltpu.VMEM((SCAN,), jnp.int32),
        pltpu.VMEM((SCAN,), jnp.int32),
        pltpu.VMEM((PENDA,), jnp.int32),
        pltpu.VMEM((PENDA,), jnp.int32),
        pltpu.VMEM((B,), jnp.int32),
        pltpu.VMEM((B,), jnp.float32),
        pltpu.VMEM((3 * B,), jnp.int32),
        pltpu.VMEM((N_WORKERS * 8,), jnp.int32),
        pltpu.SemaphoreType.DMA,
        pltpu.SemaphoreType.DMA,
        pltpu.SemaphoreType.DMA,
        pltpu.SemaphoreType.DMA,
        pltpu.SemaphoreType.DMA,
        pltpu.SemaphoreType.DMA,
    ],
    compiler_params=_params,
)
def _extract(dstE, srcE, wgtE, counts, recsP,
             d0, d1, d2, d3, idbuf, basebuf, sstage, wstage, rec, cstage,
             s0, s1, s2, s3, sem, sem2):
    c = lax.axis_index("c")
    s = lax.axis_index("s")
    g = c * 16 + s
    lo = g * RPT
    hi = lo + RPT

    pltpu.sync_copy(counts, cstage)
    myoff, _ = _region(cstage, c, s)

    iota = lax.iota(jnp.int32, 16)
    bufs = [(d0, s0), (d1, s1), (d2, s2), (d3, s3)]

    def stage_start(m, dbuf, dsem):
        off = pl.multiple_of(jnp.minimum(m, NSCAN - 1) * SCAN, SCAN)
        pltpu.make_async_copy(dstE.at[pl.ds(off, SCAN)], dbuf, dsem).start()

    def stage_wait(dbuf, dsem):
        pltpu.make_async_copy(dstE.at[pl.ds(0, SCAN)], dbuf, dsem).wait()

    for j in range(4):
        stage_start(j, *bufs[j])

    def flush(src_off, dst_batch_off):
        for j in range(BG):
            rec[pl.ds(2 * B + j * 16, 16)] = basebuf[pl.ds(src_off + j * 16, 16)]
            sstage[pl.ds(j * 16, 16)] = idbuf[pl.ds(src_off + j * 16, 16)]
        h1 = pltpu.async_copy(srcE.at[sstage], rec.at[pl.ds(0, B)], sem)
        h2 = pltpu.async_copy(wgtE.at[sstage], wstage, sem2)
        h1.wait()
        h2.wait()
        for j in range(BG):
            rec[pl.ds(B + j * 16, 16)] = plsc.bitcast(
                wstage[pl.ds(j * 16, 16)], jnp.int32)
        pltpu.sync_copy(rec, recsP.at[pl.ds(dst_batch_off, 3 * B)])

    def do_chunk(m, dscan, carry):
        cnt, nbw = carry
        cbase = m * SCAN

        def scan_step(i, cnt):
            d = dscan[pl.ds(i * 16, 16)]
            msk = (d >= lo) & (d < hi)
            ids = jnp.full((16,), cbase + i * 16, dtype=jnp.int32) + iota
            base = (d - lo) * EMB
            incl = jnp.cumsum(jnp.where(msk, 1, 0))
            pos = jnp.where(msk, cnt + incl - 1, PEND + iota)
            plsc.store_scatter(idbuf, [pos], ids)
            plsc.store_scatter(basebuf, [pos], base)
            return cnt + incl[15]

        cnt = lax.fori_loop(0, SCAN // 16, scan_step, cnt, unroll=8)

        nb = cnt // B

        def flush_body(b, _):
            dst_off = pl.multiple_of((myoff // B + nbw + b) * 3 * B, 3 * B)
            flush(b * B, dst_off)
            return 0

        lax.fori_loop(0, nb, flush_body, 0)

        rem_start = nb * B
        for j in range(BG):
            tv = idbuf[pl.ds(rem_start + j * 16, 16)]
            bv = basebuf[pl.ds(rem_start + j * 16, 16)]
            idbuf[pl.ds(j * 16, 16)] = tv
            basebuf[pl.ds(j * 16, 16)] = bv
        return (cnt - rem_start, nbw + nb)

    def quad_body(q, carry):
        m = q * 4
        for j in range(4):
            dbuf, dsem = bufs[j]
            stage_wait(dbuf, dsem)
            carry = do_chunk(m + j, dbuf, carry)
            stage_start(m + 4 + j, dbuf, dsem)
        return carry

    cnt, nbw = lax.fori_loop(0, NSCAN // 4, quad_body, (0, 0))
    for j in range(4):
        stage_wait(*bufs[j])

    @pl.when(cnt > 0)
    def _():
        sent_id = jnp.full((16,), EP - 1, dtype=jnp.int32)
        sent_base = jnp.zeros((16,), jnp.int32)
        for j in range(BG):
            idbuf[pl.ds(cnt + j * 16, 16)] = sent_id
            basebuf[pl.ds(cnt + j * 16, 16)] = sent_base
        flush(0, pl.multiple_of((myoff // B + nbw) * 3 * B, 3 * B))

    @pl.when(g == N_WORKERS - 1)
    def _():
        sent_id = jnp.full((16,), EP - 1, dtype=jnp.int32)
        sent_base = jnp.zeros((16,), jnp.int32)
        for j in range(BG):
            idbuf[pl.ds(j * 16, 16)] = sent_id
            basebuf[pl.ds(j * 16, 16)] = sent_base
        mycp_end = myoff // B + nbw + jnp.where(cnt > 0, 1, 0)
        flush(0, pl.multiple_of(mycp_end * 3 * B, 3 * B))
        flush(0, pl.multiple_of((mycp_end + 1) * 3 * B, 3 * B))


@functools.partial(
    pl.kernel,
    mesh=_mesh,
    out_type=jax.ShapeDtypeStruct((NP * EMB,), jnp.float32),
    scratch_types=[
        pltpu.VMEM((B,), jnp.int32),
        pltpu.VMEM((B,), jnp.int32),
        pltpu.VMEM((B,), jnp.int32),
        pltpu.VMEM((B,), jnp.int32),
        pltpu.VMEM((2 * B,), jnp.int32),
        pltpu.VMEM((2 * B,), jnp.int32),
        pltpu.VMEM((B, EMB), jnp.float32),
        pltpu.VMEM((B, EMB), jnp.float32),
        pltpu.VMEM((N_WORKERS * 8,), jnp.int32),
        pltpu.VMEM((ACCW,), jnp.float32),
        pltpu.SemaphoreType.DMA,
        pltpu.SemaphoreType.DMA,
        pltpu.SemaphoreType.DMA,
        pltpu.SemaphoreType.DMA,
    ],
    compiler_params=_params,
)
def _layer(tbl, recsP, counts, out,
           src_a, src_b, gsrc_a, gsrc_b, wb_a, wb_b, rows_a, rows_b,
           cstage, acc, sem_ra, sem_rb, sem_ga, sem_gb):
    c = lax.axis_index("c")
    s = lax.axis_index("s")
    g = c * 16 + s

    pltpu.sync_copy(counts, cstage)
    myoff, mycp = _region(cstage, c, s)
    b0 = myoff // B
    nb = mycp // B

    iota = lax.iota(jnp.int32, 16)
    zero16 = jnp.zeros((16,), jnp.float32)

    def zero_body(i, _):
        acc[pl.ds(i * 16, 16)] = zero16
        return 0

    lax.fori_loop(0, ACCW // 16, zero_body, 0, unroll=8)

    def start_recs(bi, sbuf, wbuf, sem):
        off = pl.multiple_of((b0 + bi) * 3 * B, 3 * B)
        cp1 = pltpu.make_async_copy(recsP.at[pl.ds(off, B)], sbuf, sem)
        cp2 = pltpu.make_async_copy(
            recsP.at[pl.ds(pl.multiple_of(off + B, 8), 2 * B)], wbuf, sem)
        cp1.start()
        cp2.start()

    def wait_recs(sbuf, wbuf, sem):
        pltpu.make_async_copy(recsP.at[pl.ds(0, B)], sbuf, sem).wait()
        pltpu.make_async_copy(recsP.at[pl.ds(0, 2 * B)], wbuf, sem).wait()

    def start_rows(sbuf, rbuf, sem):
        pltpu.make_async_copy(tbl.at[sbuf], rbuf, sem).start()

    def wait_rows(sbuf, rbuf, sem):
        pltpu.make_async_copy(tbl.at[sbuf], rbuf, sem).wait()

    def compute(wbuf, rbuf):
        def acc_body(q, _):
            e0 = q * 16
            w16 = plsc.bitcast(wbuf[pl.ds(e0, 16)], jnp.float32)
            b16 = wbuf[pl.ds(B + e0, 16)]
            for l in range(16):
                e = e0 + l
                wv = jnp.full((16,), w16[l], dtype=jnp.float32)
                bv = jnp.full((16,), b16[l], dtype=jnp.int32) + iota
                xs = [rbuf[e, pl.ds(j * 16, 16)] * wv for j in range(4)]
                ix = [bv + (j * 16) for j in range(4)]
                for j in range(4):
                    plsc.addupdate_scatter(acc, [ix[j]], xs[j])
            return 0

        lax.fori_loop(0, BG, acc_body, 0, unroll=2)

    def snap(sbuf, gbuf):
        for j in range(BG):
            gbuf[pl.ds(j * 16, 16)] = sbuf[pl.ds(j * 16, 16)]

    start_recs(0, src_a, wb_a, sem_ra)
    start_recs(1, src_b, wb_b, sem_rb)
    wait_recs(src_a, wb_a, sem_ra)
    snap(src_a, gsrc_a)
    start_rows(gsrc_a, rows_a, sem_ga)

    def pair_body(k, _):
        ba = 2 * k
        wait_recs(src_b, wb_b, sem_rb)
        snap(src_b, gsrc_b)
        start_rows(gsrc_b, rows_b, sem_gb)
        wait_rows(gsrc_a, rows_a, sem_ga)
        compute(wb_a, rows_a)
        start_recs(ba + 2, src_a, wb_a, sem_ra)
        wait_recs(src_a, wb_a, sem_ra)
        snap(src_a, gsrc_a)
        start_rows(gsrc_a, rows_a, sem_ga)
        wait_rows(gsrc_b, rows_b, sem_gb)
        compute(wb_b, rows_b)
        start_recs(ba + 3, src_b, wb_b, sem_rb)
        return 0

    lax.fori_loop(0, nb // 2, pair_body, 0)

    wait_rows(gsrc_a, rows_a, sem_ga)

    @pl.when(nb % 2 == 1)
    def _():
        compute(wb_a, rows_a)

    wait_recs(src_b, wb_b, sem_rb)

    pltpu.sync_copy(acc, out.at[pl.ds(pl.multiple_of(g * ACCW, 8), ACCW)])


_MEAN_BLK = 3128


def _mean_body(t0, t1, t2, t3, o):
    o[...] = 0.25 * ((t0[...] + t1[...]) + (t2[...] + t3[...]))


_mean = pl.pallas_call(
    _mean_body,
    grid=(NP // _MEAN_BLK,),
    in_specs=[pl.BlockSpec((_MEAN_BLK, EMB), lambda i: (i, 0))] * 4,
    out_specs=pl.BlockSpec((_MEAN_BLK, EMB), lambda i: (i, 0)),
    out_shape=jax.ShapeDtypeStruct((NP, EMB), jnp.float32),
)


def kernel(edge_index, edge_weight, user_emb, item_emb):
    rpad = jnp.zeros((NP - N, EMB), jnp.float32)
    t0 = jnp.concatenate([user_emb, item_emb, rpad], axis=0)

    pad = EP - E
    pad_idx = (jnp.arange(pad, dtype=jnp.int32) * 97) % N
    srcE = jnp.concatenate([edge_index[0], pad_idx])
    dstE = jnp.concatenate([edge_index[1], pad_idx])
    wgtE = jnp.concatenate([edge_weight, jnp.zeros((pad,), jnp.float32)])

    counts = _count(dstE)
    recsP = _extract(dstE, srcE, wgtE, counts)

    t1 = _layer(t0, recsP, counts).reshape(NP, EMB)
    t2 = _layer(t1, recsP, counts).reshape(NP, EMB)
    t3 = _layer(t2, recsP, counts).reshape(NP, EMB)

    final = _mean(t0, t1, t2, t3)
    return final[:N_U], final[N_U:N]
